# baseline (speedup 1.0000x reference)
import functools

import jax
import jax.numpy as jnp
from jax import lax
from jax.experimental import pallas as pl
from jax.experimental.pallas import tpu as pltpu
from jax.experimental.pallas import tpu_sc as plsc

NC = 2
NS = 16
NW = NC * NS
LANES = 16
CHUNK = 96
CP = 128
NBUF = 2


@functools.cache
def _build_sc_call(B, C, H, W, N):
    NPIX = H * W
    WPB = NW // B
    SPAN = ((N + WPB - 1) // WPB + CHUNK - 1) // CHUNK * CHUNK
    NFULL = SPAN // CHUNK
    LAST = N - (WPB - 1) * SPAN
    NFULL_LAST = LAST // CHUNK
    TAIL = LAST - NFULL_LAST * CHUNK
    NCH_LAST = NFULL_LAST + (1 if TAIL else 0)
    assert TAIL % LANES == 0 and LAST > 0

    mesh = plsc.VectorSubcoreMesh(core_axis_name="c", subcore_axis_name="s")

    def vmem(shape, dtype):
        return [pltpu.VMEM(shape, dtype) for _ in range(NBUF)]

    @functools.partial(
        pl.kernel,
        out_type=jax.ShapeDtypeStruct((B, N, C), jnp.float32),
        mesh=mesh,
        scratch_types=[
            vmem((CHUNK,), jnp.float32),
            vmem((CHUNK,), jnp.float32),
            vmem((CHUNK,), jnp.int32),
            vmem((CHUNK,), jnp.int32),
            vmem((CHUNK,), jnp.int32),
            vmem((CHUNK,), jnp.int32),
            vmem((CHUNK,), jnp.float32),
            vmem((CHUNK,), jnp.float32),
            vmem((CHUNK,), jnp.float32),
            vmem((CHUNK,), jnp.float32),
            vmem((CHUNK, CP), jnp.float32),
            vmem((CHUNK, CP), jnp.float32),
            vmem((CHUNK, CP), jnp.float32),
            vmem((CHUNK, CP), jnp.float32),
            vmem((CHUNK, C), jnp.float32),
            [pltpu.SemaphoreType.DMA for _ in range(NBUF)],
            [pltpu.SemaphoreType.DMA for _ in range(NBUF)],
            [pltpu.SemaphoreType.DMA for _ in range(NBUF)],
        ],
        compiler_params=pltpu.CompilerParams(needs_layout_passes=False,
                                             use_tc_tiling_on_sc=True),
    )
    def sc_sample(table, xs, ys, out, xbuf, ybuf, i00, i01, i10, i11,
                  w00, w01, w10, w11, v00, v01, v10, v11, outb,
                  vsem, gsem, osem):
        cid = lax.axis_index("c")
        sid = lax.axis_index("s")
        wid = sid * NC + cid
        wloc = lax.rem(wid, WPB)
        bidx = lax.div(wid, WPB)
        tab_off = bidx * NPIX
        vbase = bidx * N + wloc * SPAN
        obase = wloc * SPAN
        is_last = wloc == WPB - 1
        nch = jnp.where(is_last, NCH_LAST, NFULL)

        iota = lax.broadcasted_iota(jnp.int32, (LANES,), 0)
        zeros16 = jnp.zeros((LANES,), jnp.int32)
        ones16 = zeros16 + 1

        def fetch_verts(ci, s):
            pltpu.async_copy(xs.at[pl.ds(vbase + ci * CHUNK, CHUNK)],
                             xbuf[s], vsem[s])
            pltpu.async_copy(ys.at[pl.ds(vbase + ci * CHUNK, CHUNK)],
                             ybuf[s], vsem[s])

        def stage(ci, s):
            pltpu.make_async_copy(xs.at[pl.ds(0, CHUNK)], xbuf[s],
                                  vsem[s]).wait()
            pltpu.make_async_copy(ys.at[pl.ds(0, CHUNK)], ybuf[s],
                                  vsem[s]).wait()
            for g in range(CHUNK // LANES):
                sl = pl.ds(g * LANES, LANES)
                x = xbuf[s][sl]
                y = ybuf[s][sl]
                fx = jnp.minimum(jnp.maximum((x + 1.0) * 0.5 * (W - 1.0), 0.0),
                                 W - 1.0)
                fy = jnp.minimum(jnp.maximum((y + 1.0) * 0.5 * (H - 1.0), 0.0),
                                 H - 1.0)
                ix0 = fx.astype(jnp.int32)
                iy0 = fy.astype(jnp.int32)
                wx1 = fx - ix0.astype(jnp.float32)
                wy1 = fy - iy0.astype(jnp.float32)
                wx0 = 1.0 - wx1
                wy0 = 1.0 - wy1
                dx = jnp.where(ix0 < W - 1, 1, 0)
                dy = jnp.where(iy0 < H - 1, W, 0)
                p00 = tab_off + iy0 * W + ix0
                i00[s][sl] = p00
                i01[s][sl] = p00 + dx
                i10[s][sl] = p00 + dy
                i11[s][sl] = p00 + dy + dx
                w00[s][sl] = wy0 * wx0
                w01[s][sl] = wy0 * wx1
                w10[s][sl] = wy1 * wx0
                w11[s][sl] = wy1 * wx1

            @pl.when(ci + NBUF < nch)
            def _():
                fetch_verts(ci + NBUF, s)
            pltpu.async_copy(table.at[i00[s]], v00[s], gsem[s])
            pltpu.async_copy(table.at[i01[s]], v01[s], gsem[s])
            pltpu.async_copy(table.at[i10[s]], v10[s], gsem[s])
            pltpu.async_copy(table.at[i11[s]], v11[s], gsem[s])

        def compute_write(ci, s):
            for buf in (v00, v01, v10, v11):
                pltpu.make_async_copy(table.at[i00[s]], buf[s], gsem[s]).wait()

            @pl.when(ci >= NBUF)
            def _():
                pltpu.make_async_copy(out.at[0, pl.ds(0, CHUNK)], outb[s],
                                      osem[s]).wait()

            for g in range(CHUNK // LANES):
                sl = pl.ds(g * LANES, LANES)
                a00 = w00[s][sl]
                a01 = w01[s][sl]
                a10 = w10[s][sl]
                a11 = w11[s][sl]
                rows = g * LANES + iota

                def ch_body(c8, carry2, a00=a00, a01=a01, a10=a10, a11=a11,
                            rows=rows):
                    for u in range(8):
                        cvec = zeros16 + (c8 * 8 + u)
                        t = (plsc.load_gather(v00[s], [rows, cvec]) * a00
                             + plsc.load_gather(v01[s], [rows, cvec]) * a01
                             + plsc.load_gather(v10[s], [rows, cvec]) * a10
                             + plsc.load_gather(v11[s], [rows, cvec]) * a11)
                        plsc.store_scatter(outb[s], [rows, cvec], t)
                    return carry2

                lax.fori_loop(0, C // 8, ch_body, 0)
            full = jnp.logical_or(jnp.logical_not(is_last), ci < NFULL_LAST)

            @pl.when(full)
            def _():
                pltpu.async_copy(outb[s],
                                 out.at[bidx, pl.ds(obase + ci * CHUNK, CHUNK)],
                                 osem[s])

            if TAIL:
                @pl.when(jnp.logical_not(full))
                def _():
                    pltpu.sync_copy(
                        outb[s].at[pl.ds(0, TAIL)],
                        out.at[bidx, pl.ds(obase + NFULL_LAST * CHUNK, TAIL)])

        fetch_verts(0, 0)
        fetch_verts(1, 1)
        stage(0, 0)

        def pair_body(k, carry):
            i0 = 2 * k
            i1 = i0 + 1

            @pl.when(i1 < nch)
            def _():
                stage(i1, 1)

            @pl.when(i0 < nch)
            def _():
                compute_write(i0, 0)

            @pl.when(i1 < nch)
            def _():
                @pl.when(i1 + 1 < nch)
                def _():
                    stage(i1 + 1, 0)
                compute_write(i1, 1)
            return carry

        lax.fori_loop(0, (max(NFULL, NCH_LAST) + 1) // 2, pair_body, 0)

        for s in range(NBUF):
            def drain(s=s):
                pltpu.make_async_copy(out.at[0, pl.ds(0, CHUNK)], outb[s],
                                      osem[s]).wait()
            if TAIL and (NCH_LAST - 1) % NBUF == s:
                pl.when(jnp.logical_not(is_last))(drain)
            else:
                drain()

    return sc_sample


def kernel(image_features, vertices):
    B, C, H, W = image_features.shape
    N = vertices.shape[1]
    table = jnp.pad(jnp.transpose(image_features, (0, 2, 3, 1)),
                    ((0, 0), (0, 0), (0, 0), (0, CP - C)))
    table = table.reshape(B * H * W, CP)
    pad = jnp.zeros((CHUNK,), vertices.dtype)
    xs = jnp.concatenate([vertices[..., 0].reshape(-1), pad])
    ys = jnp.concatenate([vertices[..., 1].reshape(-1), pad])
    sc_sample = _build_sc_call(B, C, H, W, N)
    return sc_sample(table, xs, ys)


# speedup vs baseline: 2.7859x; 2.7859x over previous
import functools

import jax
import jax.numpy as jnp
from jax import lax
from jax.experimental import pallas as pl
from jax.experimental.pallas import tpu as pltpu
from jax.experimental.pallas import tpu_sc as plsc

NC = 2
NS = 16
NW = NC * NS
LANES = 16
CHUNK = 96
CP = 128
NBUF = 2


@functools.cache
def _build_sc_call(B, C, H, W, N):
    NPIX = H * W
    WPB = NW // B
    SPAN = ((N + WPB - 1) // WPB + CHUNK - 1) // CHUNK * CHUNK
    NFULL = SPAN // CHUNK
    LAST = N - (WPB - 1) * SPAN
    NFULL_LAST = LAST // CHUNK
    TAIL = LAST - NFULL_LAST * CHUNK
    NCH_LAST = NFULL_LAST + (1 if TAIL else 0)
    assert TAIL % LANES == 0 and LAST > 0

    mesh = plsc.VectorSubcoreMesh(core_axis_name="c", subcore_axis_name="s")

    def vmem(shape, dtype):
        return [pltpu.VMEM(shape, dtype) for _ in range(NBUF)]

    @functools.partial(
        pl.kernel,
        out_type=jax.ShapeDtypeStruct((B, N, C), jnp.float32),
        mesh=mesh,
        scratch_types=[
            vmem((CHUNK,), jnp.float32),
            vmem((CHUNK,), jnp.float32),
            vmem((CHUNK,), jnp.int32),
            vmem((CHUNK,), jnp.int32),
            vmem((CHUNK,), jnp.int32),
            vmem((CHUNK,), jnp.int32),
            vmem((CHUNK,), jnp.float32),
            vmem((CHUNK,), jnp.float32),
            vmem((CHUNK,), jnp.float32),
            vmem((CHUNK,), jnp.float32),
            vmem((CHUNK, CP), jnp.float32),
            vmem((CHUNK, CP), jnp.float32),
            vmem((CHUNK, CP), jnp.float32),
            vmem((CHUNK, CP), jnp.float32),
            vmem((CHUNK, C), jnp.float32),
            [pltpu.SemaphoreType.DMA for _ in range(NBUF)],
            [pltpu.SemaphoreType.DMA for _ in range(NBUF)],
            [pltpu.SemaphoreType.DMA for _ in range(NBUF)],
        ],
        compiler_params=pltpu.CompilerParams(needs_layout_passes=False,
                                             use_tc_tiling_on_sc=True),
    )
    def sc_sample(table, xs, ys, out, xbuf, ybuf, i00, i01, i10, i11,
                  w00, w01, w10, w11, v00, v01, v10, v11, outb,
                  vsem, gsem, osem):
        cid = lax.axis_index("c")
        sid = lax.axis_index("s")
        wid = sid * NC + cid
        wloc = lax.rem(wid, WPB)
        bidx = lax.div(wid, WPB)
        tab_off = bidx * NPIX
        vbase = bidx * N + wloc * SPAN
        obase = wloc * SPAN
        is_last = wloc == WPB - 1
        nch = jnp.where(is_last, NCH_LAST, NFULL)

        iota = lax.broadcasted_iota(jnp.int32, (LANES,), 0)
        zeros16 = jnp.zeros((LANES,), jnp.int32)
        ones16 = zeros16 + 1

        def fetch_verts(ci, s):
            pltpu.async_copy(xs.at[pl.ds(vbase + ci * CHUNK, CHUNK)],
                             xbuf[s], vsem[s])
            pltpu.async_copy(ys.at[pl.ds(vbase + ci * CHUNK, CHUNK)],
                             ybuf[s], vsem[s])

        def stage(ci, s):
            pltpu.make_async_copy(xs.at[pl.ds(0, CHUNK)], xbuf[s],
                                  vsem[s]).wait()
            pltpu.make_async_copy(ys.at[pl.ds(0, CHUNK)], ybuf[s],
                                  vsem[s]).wait()
            for g in range(CHUNK // LANES):
                sl = pl.ds(g * LANES, LANES)
                x = xbuf[s][sl]
                y = ybuf[s][sl]
                fx = jnp.minimum(jnp.maximum((x + 1.0) * 0.5 * (W - 1.0), 0.0),
                                 W - 1.0)
                fy = jnp.minimum(jnp.maximum((y + 1.0) * 0.5 * (H - 1.0), 0.0),
                                 H - 1.0)
                ix0 = fx.astype(jnp.int32)
                iy0 = fy.astype(jnp.int32)
                wx1 = fx - ix0.astype(jnp.float32)
                wy1 = fy - iy0.astype(jnp.float32)
                wx0 = 1.0 - wx1
                wy0 = 1.0 - wy1
                dx = jnp.where(ix0 < W - 1, 1, 0)
                dy = jnp.where(iy0 < H - 1, W, 0)
                p00 = tab_off + iy0 * W + ix0
                i00[s][sl] = p00
                i01[s][sl] = p00 + dx
                i10[s][sl] = p00 + dy
                i11[s][sl] = p00 + dy + dx
                w00[s][sl] = wy0 * wx0
                w01[s][sl] = wy0 * wx1
                w10[s][sl] = wy1 * wx0
                w11[s][sl] = wy1 * wx1

            @pl.when(ci + NBUF < nch)
            def _():
                fetch_verts(ci + NBUF, s)
            pltpu.async_copy(table.at[i00[s]], v00[s], gsem[s])
            pltpu.async_copy(table.at[i01[s]], v01[s], gsem[s])
            pltpu.async_copy(table.at[i10[s]], v10[s], gsem[s])
            pltpu.async_copy(table.at[i11[s]], v11[s], gsem[s])

        def compute_write(ci, s):
            for buf in (v00, v01, v10, v11):
                pltpu.make_async_copy(table.at[i00[s]], buf[s], gsem[s]).wait()

            @pl.when(ci >= NBUF)
            def _():
                pltpu.make_async_copy(out.at[0, pl.ds(0, CHUNK)], outb[s],
                                      osem[s]).wait()

            def row_body(r2, carry2):
                for u in range(2):
                    r = r2 * 2 + u
                    bidx = zeros16 + r
                    b00 = plsc.load_gather(w00[s], [bidx])
                    b01 = plsc.load_gather(w01[s], [bidx])
                    b10 = plsc.load_gather(w10[s], [bidx])
                    b11 = plsc.load_gather(w11[s], [bidx])
                    for j in range(C // LANES):
                        sl = pl.ds(j * LANES, LANES)
                        outb[s][r, sl] = (v00[s][r, sl] * b00
                                          + v01[s][r, sl] * b01
                                          + v10[s][r, sl] * b10
                                          + v11[s][r, sl] * b11)
                return carry2

            lax.fori_loop(0, CHUNK // 2, row_body, 0)
            full = jnp.logical_or(jnp.logical_not(is_last), ci < NFULL_LAST)

            @pl.when(full)
            def _():
                pltpu.async_copy(outb[s],
                                 out.at[bidx, pl.ds(obase + ci * CHUNK, CHUNK)],
                                 osem[s])

            if TAIL:
                @pl.when(jnp.logical_not(full))
                def _():
                    pltpu.sync_copy(
                        outb[s].at[pl.ds(0, TAIL)],
                        out.at[bidx, pl.ds(obase + NFULL_LAST * CHUNK, TAIL)])

        fetch_verts(0, 0)
        fetch_verts(1, 1)
        stage(0, 0)

        def pair_body(k, carry):
            i0 = 2 * k
            i1 = i0 + 1

            @pl.when(i1 < nch)
            def _():
                stage(i1, 1)

            @pl.when(i0 < nch)
            def _():
                compute_write(i0, 0)

            @pl.when(i1 < nch)
            def _():
                @pl.when(i1 + 1 < nch)
                def _():
                    stage(i1 + 1, 0)
                compute_write(i1, 1)
            return carry

        lax.fori_loop(0, (max(NFULL, NCH_LAST) + 1) // 2, pair_body, 0)

        for s in range(NBUF):
            def drain(s=s):
                pltpu.make_async_copy(out.at[0, pl.ds(0, CHUNK)], outb[s],
                                      osem[s]).wait()
            if TAIL and (NCH_LAST - 1) % NBUF == s:
                pl.when(jnp.logical_not(is_last))(drain)
            else:
                drain()

    return sc_sample


def kernel(image_features, vertices):
    B, C, H, W = image_features.shape
    N = vertices.shape[1]
    table = jnp.pad(jnp.transpose(image_features, (0, 2, 3, 1)),
                    ((0, 0), (0, 0), (0, 0), (0, CP - C)))
    table = table.reshape(B * H * W, CP)
    pad = jnp.zeros((CHUNK,), vertices.dtype)
    xs = jnp.concatenate([vertices[..., 0].reshape(-1), pad])
    ys = jnp.concatenate([vertices[..., 1].reshape(-1), pad])
    sc_sample = _build_sc_call(B, C, H, W, N)
    return sc_sample(table, xs, ys)


# speedup vs baseline: 3.4904x; 1.2529x over previous
import functools

import jax
import jax.numpy as jnp
from jax import lax
from jax.experimental import pallas as pl
from jax.experimental.pallas import tpu as pltpu
from jax.experimental.pallas import tpu_sc as plsc

NC = 2
NS = 16
NW = NC * NS
LANES = 16
CHUNK = 96
CP = 128
NBUF = 2


@functools.cache
def _build_sc_call(B, C, H, W, N):
    NPIX = H * W
    WPB = NW // B
    SPAN = ((N + WPB - 1) // WPB + CHUNK - 1) // CHUNK * CHUNK
    NFULL = SPAN // CHUNK
    LAST = N - (WPB - 1) * SPAN
    NFULL_LAST = LAST // CHUNK
    TAIL = LAST - NFULL_LAST * CHUNK
    NCH_LAST = NFULL_LAST + (1 if TAIL else 0)
    assert TAIL % LANES == 0 and LAST > 0

    mesh = plsc.VectorSubcoreMesh(core_axis_name="c", subcore_axis_name="s")

    def vmem(shape, dtype):
        return [pltpu.VMEM(shape, dtype) for _ in range(NBUF)]

    @functools.partial(
        pl.kernel,
        out_type=jax.ShapeDtypeStruct((B, N, C), jnp.float32),
        mesh=mesh,
        scratch_types=[
            vmem((CHUNK,), jnp.float32),
            vmem((CHUNK,), jnp.float32),
            vmem((CHUNK,), jnp.int32),
            vmem((CHUNK,), jnp.int32),
            vmem((CHUNK,), jnp.int32),
            vmem((CHUNK,), jnp.int32),
            vmem((CHUNK,), jnp.float32),
            vmem((CHUNK,), jnp.float32),
            vmem((CHUNK,), jnp.float32),
            vmem((CHUNK,), jnp.float32),
            vmem((CHUNK, CP), jnp.float32),
            vmem((CHUNK, CP), jnp.float32),
            vmem((CHUNK, CP), jnp.float32),
            vmem((CHUNK, CP), jnp.float32),
            vmem((CHUNK, C), jnp.float32),
            [pltpu.SemaphoreType.DMA for _ in range(NBUF)],
            [pltpu.SemaphoreType.DMA for _ in range(NBUF)],
            [pltpu.SemaphoreType.DMA for _ in range(NBUF)],
        ],
        compiler_params=pltpu.CompilerParams(needs_layout_passes=False,
                                             use_tc_tiling_on_sc=True),
    )
    def sc_sample(table, xs, ys, out, xbuf, ybuf, i00, i01, i10, i11,
                  w00, w01, w10, w11, v00, v01, v10, v11, outb,
                  vsem, gsem, osem):
        cid = lax.axis_index("c")
        sid = lax.axis_index("s")
        wid = sid * NC + cid
        wloc = lax.rem(wid, WPB)
        bidx = lax.div(wid, WPB)
        tab_off = bidx * NPIX
        vbase = bidx * N + wloc * SPAN
        obase = wloc * SPAN
        is_last = wloc == WPB - 1
        nch = jnp.where(is_last, NCH_LAST, NFULL)

        iota = lax.broadcasted_iota(jnp.int32, (LANES,), 0)
        zeros16 = jnp.zeros((LANES,), jnp.int32)
        ones16 = zeros16 + 1

        def fetch_verts(ci, s):
            pltpu.async_copy(xs.at[pl.ds(vbase + ci * CHUNK, CHUNK)],
                             xbuf[s], vsem[s])
            pltpu.async_copy(ys.at[pl.ds(vbase + ci * CHUNK, CHUNK)],
                             ybuf[s], vsem[s])

        def stage(ci, s):
            pltpu.make_async_copy(xs.at[pl.ds(0, CHUNK)], xbuf[s],
                                  vsem[s]).wait()
            pltpu.make_async_copy(ys.at[pl.ds(0, CHUNK)], ybuf[s],
                                  vsem[s]).wait()
            for g in range(CHUNK // LANES):
                sl = pl.ds(g * LANES, LANES)
                x = xbuf[s][sl]
                y = ybuf[s][sl]
                fx = jnp.minimum(jnp.maximum((x + 1.0) * 0.5 * (W - 1.0), 0.0),
                                 W - 1.0)
                fy = jnp.minimum(jnp.maximum((y + 1.0) * 0.5 * (H - 1.0), 0.0),
                                 H - 1.0)
                ix0 = fx.astype(jnp.int32)
                iy0 = fy.astype(jnp.int32)
                wx1 = fx - ix0.astype(jnp.float32)
                wy1 = fy - iy0.astype(jnp.float32)
                wx0 = 1.0 - wx1
                wy0 = 1.0 - wy1
                dx = jnp.where(ix0 < W - 1, 1, 0)
                dy = jnp.where(iy0 < H - 1, W, 0)
                p00 = tab_off + iy0 * W + ix0
                i00[s][sl] = p00
                i01[s][sl] = p00 + dx
                i10[s][sl] = p00 + dy
                i11[s][sl] = p00 + dy + dx
                w00[s][sl] = wy0 * wx0
                w01[s][sl] = wy0 * wx1
                w10[s][sl] = wy1 * wx0
                w11[s][sl] = wy1 * wx1

            @pl.when(ci + NBUF < nch)
            def _():
                fetch_verts(ci + NBUF, s)
            pltpu.async_copy(table.at[i00[s]], v00[s], gsem[s])
            pltpu.async_copy(table.at[i01[s]], v01[s], gsem[s])
            pltpu.async_copy(table.at[i10[s]], v10[s], gsem[s])
            pltpu.async_copy(table.at[i11[s]], v11[s], gsem[s])

        def compute_write(ci, s):
            for buf in (v00, v01, v10, v11):
                pltpu.make_async_copy(table.at[i00[s]], buf[s], gsem[s]).wait()

            @pl.when(ci >= NBUF)
            def _():
                pltpu.make_async_copy(out.at[0, pl.ds(0, CHUNK)], outb[s],
                                      osem[s]).wait()

            def row_body(r2, carry2):
                for u in range(4):
                    r = r2 * 4 + u
                    bidx = zeros16 + r
                    b00 = plsc.load_gather(w00[s], [bidx])
                    b01 = plsc.load_gather(w01[s], [bidx])
                    b10 = plsc.load_gather(w10[s], [bidx])
                    b11 = plsc.load_gather(w11[s], [bidx])
                    for j in range(C // LANES):
                        sl = pl.ds(j * LANES, LANES)
                        outb[s][r, sl] = (v00[s][r, sl] * b00
                                          + v01[s][r, sl] * b01
                                          + v10[s][r, sl] * b10
                                          + v11[s][r, sl] * b11)
                return carry2

            lax.fori_loop(0, CHUNK // 4, row_body, 0)
            full = jnp.logical_or(jnp.logical_not(is_last), ci < NFULL_LAST)

            @pl.when(full)
            def _():
                pltpu.async_copy(outb[s],
                                 out.at[bidx, pl.ds(obase + ci * CHUNK, CHUNK)],
                                 osem[s])

            if TAIL:
                @pl.when(jnp.logical_not(full))
                def _():
                    pltpu.sync_copy(
                        outb[s].at[pl.ds(0, TAIL)],
                        out.at[bidx, pl.ds(obase + NFULL_LAST * CHUNK, TAIL)])

        fetch_verts(0, 0)
        fetch_verts(1, 1)
        stage(0, 0)

        def pair_body(k, carry):
            i0 = 2 * k
            i1 = i0 + 1

            @pl.when(i1 < nch)
            def _():
                stage(i1, 1)

            @pl.when(i0 < nch)
            def _():
                compute_write(i0, 0)

            @pl.when(i1 < nch)
            def _():
                @pl.when(i1 + 1 < nch)
                def _():
                    stage(i1 + 1, 0)
                compute_write(i1, 1)
            return carry

        lax.fori_loop(0, (max(NFULL, NCH_LAST) + 1) // 2, pair_body, 0)

        for s in range(NBUF):
            def drain(s=s):
                pltpu.make_async_copy(out.at[0, pl.ds(0, CHUNK)], outb[s],
                                      osem[s]).wait()
            if TAIL and (NCH_LAST - 1) % NBUF == s:
                pl.when(jnp.logical_not(is_last))(drain)
            else:
                drain()

    return sc_sample


def kernel(image_features, vertices):
    B, C, H, W = image_features.shape
    N = vertices.shape[1]
    table = jnp.pad(jnp.transpose(image_features, (0, 2, 3, 1)),
                    ((0, 0), (0, 0), (0, 0), (0, CP - C)))
    table = table.reshape(B * H * W, CP)
    pad = jnp.zeros((CHUNK,), vertices.dtype)
    xs = jnp.concatenate([vertices[..., 0].reshape(-1), pad])
    ys = jnp.concatenate([vertices[..., 1].reshape(-1), pad])
    sc_sample = _build_sc_call(B, C, H, W, N)
    return sc_sample(table, xs, ys)
